# baseline (speedup 1.0000x reference)
import jax
import jax.numpy as jnp
from jax import lax
from jax.experimental import pallas as pl
from jax.experimental.pallas import tpu as pltpu
from jax.experimental.pallas import tpu_sc as plsc

N = 10000
E = 320000
D = 128
NCORES = 2
NSUB = 16
NW = NCORES * NSUB
EPT = E // NW
B = 80
NB = EPT // B
CHUNK = 25
RPT = 624

_MESH = plsc.VectorSubcoreMesh(
    core_axis_name="c", subcore_axis_name="s", num_cores=NCORES, num_subcores=NSUB
)


def _deg_body(dst3, w3, out, dacc, dstb, wb, zb):
    c = lax.axis_index("c")
    s = lax.axis_index("s")
    wid = c * NSUB + s

    def zloop(i, carry):
        zb[pl.ds(i * 16, 16)] = jnp.zeros((16,), jnp.float32)
        return carry

    lax.fori_loop(0, 40, zloop, 0)
    base = s * RPT
    pltpu.sync_copy(zb.at[pl.ds(0, RPT)], dacc.at[pl.ds(base, RPT)])

    @pl.when(s == NSUB - 1)
    def _():
        pltpu.sync_copy(zb.at[pl.ds(0, 16)], dacc.at[pl.ds(NSUB * RPT, 16)])

    plsc.subcore_barrier()
    pltpu.sync_copy(dst3.at[wid], dstb)
    pltpu.sync_copy(w3.at[wid], wb)

    def batch(j, carry):
        pltpu.sync_copy(wb.at[j], dacc.at[dstb.at[j]], add=True)
        return carry

    lax.fori_loop(0, NB, batch, 0)
    plsc.subcore_barrier()
    pltpu.sync_copy(dacc.at[pl.ds(base, RPT)], zb.at[pl.ds(0, RPT)])
    pltpu.sync_copy(zb.at[pl.ds(0, RPT)], out.at[pl.ds(c * N + base, RPT)])

    @pl.when(s == NSUB - 1)
    def _():
        pltpu.sync_copy(dacc.at[pl.ds(NSUB * RPT, 16)], zb.at[pl.ds(0, 16)])
        pltpu.sync_copy(zb.at[pl.ds(0, 16)], out.at[pl.ds(c * N + NSUB * RPT, 16)])


_deg_call = pl.kernel(
    _deg_body,
    out_type=jax.ShapeDtypeStruct((NCORES * N,), jnp.float32),
    mesh=_MESH,
    scratch_types=[
        pltpu.VMEM_SHARED((N,), jnp.float32),
        pltpu.VMEM((NB, B), jnp.int32),
        pltpu.VMEM((NB, B), jnp.float32),
        pltpu.VMEM((640,), jnp.float32),
    ],
)


def _msg_body(g, src4, dst4, w4, out, acc, srcb, dstb, wb, rows, sg, ss, sr):
    c = lax.axis_index("c")
    s = lax.axis_index("s")
    wid = c * NSUB + s

    def zloop(i, carry):
        for q in range(8):
            rows[0, i, pl.ds(q * 16, 16)] = jnp.zeros((16,), jnp.float32)
        return carry

    lax.fori_loop(0, 80, zloop, 0)
    base = s * RPT
    for k in range(7):
        pltpu.sync_copy(rows.at[0], acc.at[pl.ds(base + k * 80, 80)])
    pltpu.sync_copy(rows.at[0, pl.ds(0, 64)], acc.at[pl.ds(base + 560, 64)])

    @pl.when(s == NSUB - 1)
    def _():
        pltpu.sync_copy(rows.at[0, pl.ds(0, 16)], acc.at[pl.ds(NSUB * RPT, 16)])

    plsc.subcore_barrier()

    pltpu.sync_copy(src4.at[wid, 0], srcb.at[0])
    pltpu.sync_copy(dst4.at[wid, 0], dstb.at[0])
    pltpu.sync_copy(w4.at[wid, 0], wb.at[0])
    pltpu.async_copy(g.at[srcb.at[0, 0]], rows.at[0], sg)

    def _half(bufA, bufB, j, prefetch):
        cc = j // CHUNK
        jj = j % CHUNK
        par = cc % 2

        @pl.when(jnp.logical_and(jj == 0, j < (NB - CHUNK)))
        def _():
            cn = cc + 1
            pn = cn % 2
            pltpu.async_copy(src4.at[wid, cn], srcb.at[pn], sr)
            pltpu.async_copy(dst4.at[wid, cn], dstb.at[pn], sr)
            pltpu.async_copy(w4.at[wid, cn], wb.at[pn], sr)

        pltpu.make_async_copy(g.at[srcb.at[par, jj]], rows.at[bufA], sg).wait()

        @pl.when(j >= 1)
        def _():
            parm = ((j - 1) // CHUNK) % 2
            jm = (j - 1) % CHUNK
            pltpu.make_async_copy(rows.at[bufB], acc.at[dstb.at[parm, jm]], ss).wait()

        if prefetch:
            @pl.when(jnp.logical_and(jj == CHUNK - 1, j < NB - 1))
            def _():
                cn = cc + 1
                pn = cn % 2
                pltpu.make_async_copy(src4.at[wid, cn], srcb.at[pn], sr).wait()
                pltpu.make_async_copy(dst4.at[wid, cn], dstb.at[pn], sr).wait()
                pltpu.make_async_copy(w4.at[wid, cn], wb.at[pn], sr).wait()

            jn = j + 1
            parn = (jn // CHUNK) % 2
            jjn = jn % CHUNK
            pltpu.async_copy(g.at[srcb.at[parn, jjn]], rows.at[bufB], sg)

        for gr in range(B // 16):
            wv16 = wb[par, jj, pl.ds(gr * 16, 16)]
            for rr in range(16):
                wv = wv16[rr]
                r = gr * 16 + rr
                for q in range(8):
                    sl = pl.ds(q * 16, 16)
                    rows[bufA, r, sl] = rows[bufA, r, sl] * wv
        pltpu.async_copy(rows.at[bufA], acc.at[dstb.at[par, jj]], ss, add=True)

    def body(t, carry):
        _half(0, 1, 2 * t, True)
        _half(1, 0, 2 * t + 1, True)
        return carry

    lax.fori_loop(0, (NB - 1) // 2, body, 0)
    _half(0, 1, NB - 1, False)
    pltpu.make_async_copy(
        rows.at[0], acc.at[dstb.at[((NB - 1) // CHUNK) % 2, (NB - 1) % CHUNK]], ss
    ).wait()
    plsc.subcore_barrier()
    for k in range(7):
        pltpu.sync_copy(acc.at[pl.ds(base + k * 80, 80)], rows.at[0])
        pltpu.sync_copy(rows.at[0], out.at[c, pl.ds(base + k * 80, 80)])
    pltpu.sync_copy(acc.at[pl.ds(base + 560, 64)], rows.at[0, pl.ds(0, 64)])
    pltpu.sync_copy(rows.at[0, pl.ds(0, 64)], out.at[c, pl.ds(base + 560, 64)])

    @pl.when(s == NSUB - 1)
    def _():
        pltpu.sync_copy(acc.at[pl.ds(NSUB * RPT, 16)], rows.at[0, pl.ds(0, 16)])
        pltpu.sync_copy(rows.at[0, pl.ds(0, 16)], out.at[c, pl.ds(NSUB * RPT, 16)])


_msg_call = pl.kernel(
    _msg_body,
    out_type=jax.ShapeDtypeStruct((NCORES, N, D), jnp.float32),
    mesh=_MESH,
    scratch_types=[
        pltpu.VMEM_SHARED((N, D), jnp.float32),
        pltpu.VMEM((2, CHUNK, B), jnp.int32),
        pltpu.VMEM((2, CHUNK, B), jnp.int32),
        pltpu.VMEM((2, CHUNK, B), jnp.float32),
        pltpu.VMEM((2, B, D), jnp.float32),
        pltpu.SemaphoreType.DMA,
        pltpu.SemaphoreType.DMA,
        pltpu.SemaphoreType.DMA,
    ],
)


def _dis_body(deg_ref, dis_ref):
    dis_ref[...] = lax.rsqrt(deg_ref[0, :] + deg_ref[1, :] + 1.0)


def _gmm_body(x_ref, w_ref, disb_ref, g_ref):
    g_ref[...] = disb_ref[...] * jnp.dot(
        x_ref[...], w_ref[...], preferred_element_type=jnp.float32
    )


def _layer_body(s_ref, gp_ref, disb_ref, b_ref, w_ref, g_ref):
    t = s_ref[0] + s_ref[1] + gp_ref[...]
    z = jnp.maximum(disb_ref[...] * t + b_ref[...], 0.0)
    g_ref[...] = disb_ref[...] * jnp.dot(
        z, w_ref[...], preferred_element_type=jnp.float32
    )


def _final_body(s_ref, gp_ref, disb_ref, b_ref, wc_ref, bc_ref, out_ref):
    t = s_ref[0] + s_ref[1] + gp_ref[...]
    z = jnp.maximum(disb_ref[...] * t + b_ref[...], 0.0)
    logits = jnp.dot(z, wc_ref[...], preferred_element_type=jnp.float32) + bc_ref[...]
    m = jnp.max(logits, axis=-1, keepdims=True)
    e = jnp.exp(logits - m)
    out_ref[...] = e / jnp.sum(e, axis=-1, keepdims=True)


def kernel(x, edge_index, edge_weight, W1, b1, W2, b2, W3, b3, Wc, bc):
    src4 = edge_index[0].reshape(NW, NB // CHUNK, CHUNK, B)
    dst4 = edge_index[1].reshape(NW, NB // CHUNK, CHUNK, B)
    w4 = edge_weight.reshape(NW, NB // CHUNK, CHUNK, B)
    dst3 = dst4.reshape(NW, NB, B)
    w3 = w4.reshape(NW, NB, B)

    deg2 = _deg_call(dst3, w3).reshape(NCORES, N)
    dis = pl.pallas_call(
        _dis_body, out_shape=jax.ShapeDtypeStruct((N,), jnp.float32)
    )(deg2)
    disB = jnp.broadcast_to(dis[:, None], (N, D))

    g = pl.pallas_call(
        _gmm_body, out_shape=jax.ShapeDtypeStruct((N, D), jnp.float32)
    )(x, W1, disB)

    S = _msg_call(g, src4, dst4, w4)
    g = pl.pallas_call(
        _layer_body, out_shape=jax.ShapeDtypeStruct((N, D), jnp.float32)
    )(S, g, disB, b1, W2)

    S = _msg_call(g, src4, dst4, w4)
    g = pl.pallas_call(
        _layer_body, out_shape=jax.ShapeDtypeStruct((N, D), jnp.float32)
    )(S, g, disB, b2, W3)

    S = _msg_call(g, src4, dst4, w4)
    out = pl.pallas_call(
        _final_body, out_shape=jax.ShapeDtypeStruct((N, NC := bc.shape[0]), jnp.float32)
    )(S, g, disB, b3, Wc, bc)
    return out


# speedup vs baseline: 1.2576x; 1.2576x over previous
import jax
import jax.numpy as jnp
from jax import lax
from jax.experimental import pallas as pl
from jax.experimental.pallas import tpu as pltpu
from jax.experimental.pallas import tpu_sc as plsc

N = 10000
E = 320000
D = 128
NCORES = 2
NSUB = 16
NW = NCORES * NSUB
EPT = E // NW
B = 80
NB = EPT // B
CHUNK = 5
RPT = 624

_MESH = plsc.VectorSubcoreMesh(
    core_axis_name="c", subcore_axis_name="s", num_cores=NCORES, num_subcores=NSUB
)


def _deg_body(dst3, w3, out, dacc, dstb, wb, zb):
    c = lax.axis_index("c")
    s = lax.axis_index("s")
    wid = c * NSUB + s

    def zloop(i, carry):
        zb[pl.ds(i * 16, 16)] = jnp.zeros((16,), jnp.float32)
        return carry

    lax.fori_loop(0, 40, zloop, 0)
    base = s * RPT
    pltpu.sync_copy(zb.at[pl.ds(0, RPT)], dacc.at[pl.ds(base, RPT)])

    @pl.when(s == NSUB - 1)
    def _():
        pltpu.sync_copy(zb.at[pl.ds(0, 16)], dacc.at[pl.ds(NSUB * RPT, 16)])

    plsc.subcore_barrier()
    pltpu.sync_copy(dst3.at[wid], dstb)
    pltpu.sync_copy(w3.at[wid], wb)

    def batch(j, carry):
        pltpu.sync_copy(wb.at[j], dacc.at[dstb.at[j]], add=True)
        return carry

    lax.fori_loop(0, NB, batch, 0)
    plsc.subcore_barrier()
    pltpu.sync_copy(dacc.at[pl.ds(base, RPT)], zb.at[pl.ds(0, RPT)])
    pltpu.sync_copy(zb.at[pl.ds(0, RPT)], out.at[pl.ds(c * N + base, RPT)])

    @pl.when(s == NSUB - 1)
    def _():
        pltpu.sync_copy(dacc.at[pl.ds(NSUB * RPT, 16)], zb.at[pl.ds(0, 16)])
        pltpu.sync_copy(zb.at[pl.ds(0, 16)], out.at[pl.ds(c * N + NSUB * RPT, 16)])


_deg_call = pl.kernel(
    _deg_body,
    out_type=jax.ShapeDtypeStruct((NCORES * N,), jnp.float32),
    mesh=_MESH,
    scratch_types=[
        pltpu.VMEM_SHARED((N,), jnp.float32),
        pltpu.VMEM((NB, B), jnp.int32),
        pltpu.VMEM((NB, B), jnp.float32),
        pltpu.VMEM((640,), jnp.float32),
    ],
)


def _msg_body(g, src4, dst4, w4, out, acc, srcb, dstb, wb, dsti, rows, sg, ss, sr):
    c = lax.axis_index("c")
    s = lax.axis_index("s")
    wid = c * NSUB + s

    def zloop(i, carry):
        for q in range(8):
            rows[0, i, pl.ds(q * 16, 16)] = jnp.zeros((16,), jnp.float32)
        return carry

    lax.fori_loop(0, 80, zloop, 0)
    base = s * RPT
    for k in range(7):
        pltpu.sync_copy(rows.at[0], acc.at[pl.ds(base + k * 80, 80)])
    pltpu.sync_copy(rows.at[0, pl.ds(0, 64)], acc.at[pl.ds(base + 560, 64)])

    @pl.when(s == NSUB - 1)
    def _():
        pltpu.sync_copy(rows.at[0, pl.ds(0, 16)], acc.at[pl.ds(NSUB * RPT, 16)])

    plsc.subcore_barrier()

    pltpu.sync_copy(src4.at[wid, 0], srcb.at[0])
    pltpu.sync_copy(dst4.at[wid, 0], dstb.at[0])
    pltpu.sync_copy(w4.at[wid, 0], wb.at[0])
    pltpu.async_copy(g.at[srcb.at[0, 0]], rows.at[0], sg)
    pltpu.async_copy(g.at[srcb.at[0, 1]], rows.at[1], sg)

    def _do(j, q, prefetch):
        cc = j // CHUNK
        jj = j % CHUNK
        par = cc % 2

        @pl.when(jnp.logical_and(jj == 0, j < (NB - CHUNK)))
        def _():
            cn = cc + 1
            pn = cn % 2
            pltpu.async_copy(src4.at[wid, cn], srcb.at[pn], sr)
            pltpu.async_copy(dst4.at[wid, cn], dstb.at[pn], sr)
            pltpu.async_copy(w4.at[wid, cn], wb.at[pn], sr)

        pltpu.make_async_copy(g.at[srcb.at[par, jj]], rows.at[q], sg).wait()

        @pl.when(j >= 2)
        def _():
            pltpu.make_async_copy(
                rows.at[(q - 2) % 4], acc.at[dsti.at[(q - 2) % 4]], ss
            ).wait()

        if prefetch:
            @pl.when(jnp.logical_and(jj == CHUNK - 2, j < NB - 2))
            def _():
                cn = cc + 1
                pn = cn % 2
                pltpu.make_async_copy(src4.at[wid, cn], srcb.at[pn], sr).wait()
                pltpu.make_async_copy(dst4.at[wid, cn], dstb.at[pn], sr).wait()
                pltpu.make_async_copy(w4.at[wid, cn], wb.at[pn], sr).wait()

            @pl.when(j < NB - 2)
            def _():
                jn = j + 2
                parn = (jn // CHUNK) % 2
                jjn = jn % CHUNK
                pltpu.async_copy(g.at[srcb.at[parn, jjn]], rows.at[(q + 2) % 4], sg)

        for gr in range(B // 16):
            dsti[q, pl.ds(gr * 16, 16)] = dstb[par, jj, pl.ds(gr * 16, 16)]

        def grouploop(gr, c2):
            wv16 = wb[par, jj, pl.ds(gr * 16, 16)]
            for rr in range(16):
                wv = wv16[rr]
                r = gr * 16 + rr
                for qq in range(8):
                    sl = pl.ds(qq * 16, 16)
                    rows[q, r, sl] = rows[q, r, sl] * wv
            return c2

        lax.fori_loop(0, B // 16, grouploop, 0)
        pltpu.async_copy(rows.at[q], acc.at[dsti.at[q]], ss, add=True)

    def body(t, carry):
        j0 = 4 * t
        for q in range(4):
            _do(j0 + q, q, True)
        return carry

    lax.fori_loop(0, (NB - 1) // 4, body, 0)
    _do(NB - 1, 0, False)
    pltpu.make_async_copy(rows.at[3], acc.at[dsti.at[3]], ss).wait()
    pltpu.make_async_copy(rows.at[0], acc.at[dsti.at[0]], ss).wait()
    plsc.subcore_barrier()
    for k in range(7):
        pltpu.sync_copy(acc.at[pl.ds(base + k * 80, 80)], rows.at[0])
        pltpu.sync_copy(rows.at[0], out.at[c, pl.ds(base + k * 80, 80)])
    pltpu.sync_copy(acc.at[pl.ds(base + 560, 64)], rows.at[0, pl.ds(0, 64)])
    pltpu.sync_copy(rows.at[0, pl.ds(0, 64)], out.at[c, pl.ds(base + 560, 64)])

    @pl.when(s == NSUB - 1)
    def _():
        pltpu.sync_copy(acc.at[pl.ds(NSUB * RPT, 16)], rows.at[0, pl.ds(0, 16)])
        pltpu.sync_copy(rows.at[0, pl.ds(0, 16)], out.at[c, pl.ds(NSUB * RPT, 16)])


_msg_call = pl.kernel(
    _msg_body,
    out_type=jax.ShapeDtypeStruct((NCORES, N, D), jnp.float32),
    mesh=_MESH,
    scratch_types=[
        pltpu.VMEM_SHARED((N, D), jnp.float32),
        pltpu.VMEM((2, CHUNK, B), jnp.int32),
        pltpu.VMEM((2, CHUNK, B), jnp.int32),
        pltpu.VMEM((2, CHUNK, B), jnp.float32),
        pltpu.VMEM((4, B), jnp.int32),
        pltpu.VMEM((4, B, D), jnp.float32),
        pltpu.SemaphoreType.DMA,
        pltpu.SemaphoreType.DMA,
        pltpu.SemaphoreType.DMA,
    ],
)


def _dis_body(deg_ref, dis_ref):
    dis_ref[...] = lax.rsqrt(deg_ref[0, :] + deg_ref[1, :] + 1.0)


def _gmm_body(x_ref, w_ref, disb_ref, g_ref):
    g_ref[...] = disb_ref[...] * jnp.dot(
        x_ref[...], w_ref[...], preferred_element_type=jnp.float32
    )


def _layer_body(s_ref, gp_ref, disb_ref, b_ref, w_ref, g_ref):
    t = s_ref[0] + s_ref[1] + gp_ref[...]
    z = jnp.maximum(disb_ref[...] * t + b_ref[...], 0.0)
    g_ref[...] = disb_ref[...] * jnp.dot(
        z, w_ref[...], preferred_element_type=jnp.float32
    )


def _final_body(s_ref, gp_ref, disb_ref, b_ref, wc_ref, bc_ref, out_ref):
    t = s_ref[0] + s_ref[1] + gp_ref[...]
    z = jnp.maximum(disb_ref[...] * t + b_ref[...], 0.0)
    logits = jnp.dot(z, wc_ref[...], preferred_element_type=jnp.float32) + bc_ref[...]
    m = jnp.max(logits, axis=-1, keepdims=True)
    e = jnp.exp(logits - m)
    out_ref[...] = e / jnp.sum(e, axis=-1, keepdims=True)


def kernel(x, edge_index, edge_weight, W1, b1, W2, b2, W3, b3, Wc, bc):
    src4 = edge_index[0].reshape(NW, NB // CHUNK, CHUNK, B)
    dst4 = edge_index[1].reshape(NW, NB // CHUNK, CHUNK, B)
    w4 = edge_weight.reshape(NW, NB // CHUNK, CHUNK, B)
    dst3 = dst4.reshape(NW, NB, B)
    w3 = w4.reshape(NW, NB, B)

    deg2 = _deg_call(dst3, w3).reshape(NCORES, N)
    dis = pl.pallas_call(
        _dis_body, out_shape=jax.ShapeDtypeStruct((N,), jnp.float32)
    )(deg2)
    disB = jnp.broadcast_to(dis[:, None], (N, D))

    g = pl.pallas_call(
        _gmm_body, out_shape=jax.ShapeDtypeStruct((N, D), jnp.float32)
    )(x, W1, disB)

    S = _msg_call(g, src4, dst4, w4)
    g = pl.pallas_call(
        _layer_body, out_shape=jax.ShapeDtypeStruct((N, D), jnp.float32)
    )(S, g, disB, b1, W2)

    S = _msg_call(g, src4, dst4, w4)
    g = pl.pallas_call(
        _layer_body, out_shape=jax.ShapeDtypeStruct((N, D), jnp.float32)
    )(S, g, disB, b2, W3)

    S = _msg_call(g, src4, dst4, w4)
    out = pl.pallas_call(
        _final_body, out_shape=jax.ShapeDtypeStruct((N, NC := bc.shape[0]), jnp.float32)
    )(S, g, disB, b3, Wc, bc)
    return out


# speedup vs baseline: 1.2607x; 1.0025x over previous
import jax
import jax.numpy as jnp
from jax import lax
from jax.experimental import pallas as pl
from jax.experimental.pallas import tpu as pltpu
from jax.experimental.pallas import tpu_sc as plsc

N = 10000
E = 320000
D = 128
NCORES = 2
NSUB = 16
NW = NCORES * NSUB
EPT = E // NW
B = 80
NB = EPT // B
CHUNK = 5
RPT = 624

_MESH = plsc.VectorSubcoreMesh(
    core_axis_name="c", subcore_axis_name="s", num_cores=NCORES, num_subcores=NSUB
)


def _deg_body(dst3, w3, out, dacc, dstb, wb, zb):
    c = lax.axis_index("c")
    s = lax.axis_index("s")
    wid = c * NSUB + s

    def zloop(i, carry):
        zb[pl.ds(i * 16, 16)] = jnp.zeros((16,), jnp.float32)
        return carry

    lax.fori_loop(0, 40, zloop, 0)
    base = s * RPT
    pltpu.sync_copy(zb.at[pl.ds(0, RPT)], dacc.at[pl.ds(base, RPT)])

    @pl.when(s == NSUB - 1)
    def _():
        pltpu.sync_copy(zb.at[pl.ds(0, 16)], dacc.at[pl.ds(NSUB * RPT, 16)])

    plsc.subcore_barrier()
    pltpu.sync_copy(dst3.at[wid], dstb)
    pltpu.sync_copy(w3.at[wid], wb)

    def batch(j, carry):
        pltpu.sync_copy(wb.at[j], dacc.at[dstb.at[j]], add=True)
        return carry

    lax.fori_loop(0, NB, batch, 0)
    plsc.subcore_barrier()
    pltpu.sync_copy(dacc.at[pl.ds(base, RPT)], zb.at[pl.ds(0, RPT)])
    pltpu.sync_copy(zb.at[pl.ds(0, RPT)], out.at[pl.ds(c * N + base, RPT)])

    @pl.when(s == NSUB - 1)
    def _():
        pltpu.sync_copy(dacc.at[pl.ds(NSUB * RPT, 16)], zb.at[pl.ds(0, 16)])
        pltpu.sync_copy(zb.at[pl.ds(0, 16)], out.at[pl.ds(c * N + NSUB * RPT, 16)])


_deg_call = pl.kernel(
    _deg_body,
    out_type=jax.ShapeDtypeStruct((NCORES * N,), jnp.float32),
    mesh=_MESH,
    scratch_types=[
        pltpu.VMEM_SHARED((N,), jnp.float32),
        pltpu.VMEM((NB, B), jnp.int32),
        pltpu.VMEM((NB, B), jnp.float32),
        pltpu.VMEM((640,), jnp.float32),
    ],
)


def _msg_body(g, src4, dst4, w4, out, acc, srcb, dstb, wb, dsti, rows, sg, ss, sr):
    c = lax.axis_index("c")
    s = lax.axis_index("s")
    wid = c * NSUB + s

    def zloop(i, carry):
        for q in range(8):
            rows[0, i, pl.ds(q * 16, 16)] = jnp.zeros((16,), jnp.float32)
        return carry

    lax.fori_loop(0, 80, zloop, 0)
    base = s * RPT
    for k in range(7):
        pltpu.sync_copy(rows.at[0], acc.at[pl.ds(base + k * 80, 80)])
    pltpu.sync_copy(rows.at[0, pl.ds(0, 64)], acc.at[pl.ds(base + 560, 64)])

    @pl.when(s == NSUB - 1)
    def _():
        pltpu.sync_copy(rows.at[0, pl.ds(0, 16)], acc.at[pl.ds(NSUB * RPT, 16)])

    plsc.subcore_barrier()

    pltpu.sync_copy(src4.at[wid, 0], srcb.at[0])
    pltpu.sync_copy(dst4.at[wid, 0], dstb.at[0])
    pltpu.sync_copy(w4.at[wid, 0], wb.at[0])
    pltpu.async_copy(g.at[srcb.at[0, 0, pl.ds(0, 40)]], rows.at[0, pl.ds(0, 40)], sg)
    pltpu.async_copy(g.at[srcb.at[0, 0, pl.ds(40, 40)]], rows.at[0, pl.ds(40, 40)], sg)
    pltpu.async_copy(g.at[srcb.at[0, 1, pl.ds(0, 40)]], rows.at[1, pl.ds(0, 40)], sg)
    pltpu.async_copy(g.at[srcb.at[0, 1, pl.ds(40, 40)]], rows.at[1, pl.ds(40, 40)], sg)

    def _do(j, q, prefetch):
        cc = j // CHUNK
        jj = j % CHUNK
        par = cc % 2

        @pl.when(jnp.logical_and(jj == 0, j < (NB - CHUNK)))
        def _():
            cn = cc + 1
            pn = cn % 2
            pltpu.async_copy(src4.at[wid, cn], srcb.at[pn], sr)
            pltpu.async_copy(dst4.at[wid, cn], dstb.at[pn], sr)
            pltpu.async_copy(w4.at[wid, cn], wb.at[pn], sr)

        pltpu.make_async_copy(
            g.at[srcb.at[par, jj, pl.ds(0, 40)]], rows.at[q, pl.ds(0, 40)], sg
        ).wait()
        pltpu.make_async_copy(
            g.at[srcb.at[par, jj, pl.ds(40, 40)]], rows.at[q, pl.ds(40, 40)], sg
        ).wait()

        @pl.when(j >= 2)
        def _():
            pltpu.make_async_copy(
                rows.at[(q - 2) % 4], acc.at[dsti.at[(q - 2) % 4]], ss
            ).wait()

        if prefetch:
            @pl.when(jnp.logical_and(jj == CHUNK - 2, j < NB - 2))
            def _():
                cn = cc + 1
                pn = cn % 2
                pltpu.make_async_copy(src4.at[wid, cn], srcb.at[pn], sr).wait()
                pltpu.make_async_copy(dst4.at[wid, cn], dstb.at[pn], sr).wait()
                pltpu.make_async_copy(w4.at[wid, cn], wb.at[pn], sr).wait()

            @pl.when(j < NB - 2)
            def _():
                jn = j + 2
                parn = (jn // CHUNK) % 2
                jjn = jn % CHUNK
                pltpu.async_copy(
                    g.at[srcb.at[parn, jjn, pl.ds(0, 40)]],
                    rows.at[(q + 2) % 4, pl.ds(0, 40)], sg,
                )
                pltpu.async_copy(
                    g.at[srcb.at[parn, jjn, pl.ds(40, 40)]],
                    rows.at[(q + 2) % 4, pl.ds(40, 40)], sg,
                )

        for gr in range(B // 16):
            dsti[q, pl.ds(gr * 16, 16)] = dstb[par, jj, pl.ds(gr * 16, 16)]

        def grouploop(gr, c2):
            wv16 = wb[par, jj, pl.ds(gr * 16, 16)]
            for rr in range(16):
                wv = wv16[rr]
                r = gr * 16 + rr
                for qq in range(8):
                    sl = pl.ds(qq * 16, 16)
                    rows[q, r, sl] = rows[q, r, sl] * wv
            return c2

        lax.fori_loop(0, B // 16, grouploop, 0)
        pltpu.async_copy(rows.at[q], acc.at[dsti.at[q]], ss, add=True)

    def body(t, carry):
        j0 = 4 * t
        for q in range(4):
            _do(j0 + q, q, True)
        return carry

    lax.fori_loop(0, (NB - 1) // 4, body, 0)
    _do(NB - 1, 0, False)
    pltpu.make_async_copy(rows.at[3], acc.at[dsti.at[3]], ss).wait()
    pltpu.make_async_copy(rows.at[0], acc.at[dsti.at[0]], ss).wait()
    plsc.subcore_barrier()
    for k in range(7):
        pltpu.sync_copy(acc.at[pl.ds(base + k * 80, 80)], rows.at[0])
        pltpu.sync_copy(rows.at[0], out.at[c, pl.ds(base + k * 80, 80)])
    pltpu.sync_copy(acc.at[pl.ds(base + 560, 64)], rows.at[0, pl.ds(0, 64)])
    pltpu.sync_copy(rows.at[0, pl.ds(0, 64)], out.at[c, pl.ds(base + 560, 64)])

    @pl.when(s == NSUB - 1)
    def _():
        pltpu.sync_copy(acc.at[pl.ds(NSUB * RPT, 16)], rows.at[0, pl.ds(0, 16)])
        pltpu.sync_copy(rows.at[0, pl.ds(0, 16)], out.at[c, pl.ds(NSUB * RPT, 16)])


_msg_call = pl.kernel(
    _msg_body,
    out_type=jax.ShapeDtypeStruct((NCORES, N, D), jnp.float32),
    mesh=_MESH,
    scratch_types=[
        pltpu.VMEM_SHARED((N, D), jnp.float32),
        pltpu.VMEM((2, CHUNK, B), jnp.int32),
        pltpu.VMEM((2, CHUNK, B), jnp.int32),
        pltpu.VMEM((2, CHUNK, B), jnp.float32),
        pltpu.VMEM((4, B), jnp.int32),
        pltpu.VMEM((4, B, D), jnp.float32),
        pltpu.SemaphoreType.DMA,
        pltpu.SemaphoreType.DMA,
        pltpu.SemaphoreType.DMA,
    ],
)


def _dis_body(deg_ref, dis_ref):
    dis_ref[...] = lax.rsqrt(deg_ref[0, :] + deg_ref[1, :] + 1.0)


def _gmm_body(x_ref, w_ref, disb_ref, g_ref):
    g_ref[...] = disb_ref[...] * jnp.dot(
        x_ref[...], w_ref[...], preferred_element_type=jnp.float32
    )


def _layer_body(s_ref, gp_ref, disb_ref, b_ref, w_ref, g_ref):
    t = s_ref[0] + s_ref[1] + gp_ref[...]
    z = jnp.maximum(disb_ref[...] * t + b_ref[...], 0.0)
    g_ref[...] = disb_ref[...] * jnp.dot(
        z, w_ref[...], preferred_element_type=jnp.float32
    )


def _final_body(s_ref, gp_ref, disb_ref, b_ref, wc_ref, bc_ref, out_ref):
    t = s_ref[0] + s_ref[1] + gp_ref[...]
    z = jnp.maximum(disb_ref[...] * t + b_ref[...], 0.0)
    logits = jnp.dot(z, wc_ref[...], preferred_element_type=jnp.float32) + bc_ref[...]
    m = jnp.max(logits, axis=-1, keepdims=True)
    e = jnp.exp(logits - m)
    out_ref[...] = e / jnp.sum(e, axis=-1, keepdims=True)


def kernel(x, edge_index, edge_weight, W1, b1, W2, b2, W3, b3, Wc, bc):
    src4 = edge_index[0].reshape(NW, NB // CHUNK, CHUNK, B)
    dst4 = edge_index[1].reshape(NW, NB // CHUNK, CHUNK, B)
    w4 = edge_weight.reshape(NW, NB // CHUNK, CHUNK, B)
    dst3 = dst4.reshape(NW, NB, B)
    w3 = w4.reshape(NW, NB, B)

    deg2 = _deg_call(dst3, w3).reshape(NCORES, N)
    dis = pl.pallas_call(
        _dis_body, out_shape=jax.ShapeDtypeStruct((N,), jnp.float32)
    )(deg2)
    disB = jnp.broadcast_to(dis[:, None], (N, D))

    g = pl.pallas_call(
        _gmm_body, out_shape=jax.ShapeDtypeStruct((N, D), jnp.float32)
    )(x, W1, disB)

    S = _msg_call(g, src4, dst4, w4)
    g = pl.pallas_call(
        _layer_body, out_shape=jax.ShapeDtypeStruct((N, D), jnp.float32)
    )(S, g, disB, b1, W2)

    S = _msg_call(g, src4, dst4, w4)
    g = pl.pallas_call(
        _layer_body, out_shape=jax.ShapeDtypeStruct((N, D), jnp.float32)
    )(S, g, disB, b2, W3)

    S = _msg_call(g, src4, dst4, w4)
    out = pl.pallas_call(
        _final_body, out_shape=jax.ShapeDtypeStruct((N, NC := bc.shape[0]), jnp.float32)
    )(S, g, disB, b3, Wc, bc)
    return out
